# baseline (speedup 1.0000x reference)
import functools

import jax
import jax.numpy as jnp
from jax import lax
from jax.experimental import pallas as pl
from jax.experimental.pallas import tpu as pltpu
from jax.experimental.pallas import tpu_sc as plsc

B = 1024
L = 512
E = 128
VOCAB = 100000

NC = 2
NS = 16
NW = NC * NS
BPW = B // NW
CL = 128
NLC = L // CL
LANES = 16
EV = E // LANES


def _emb_body(seq_hbm, seg_hbm, tok_hbm, pos_hbm, segtab_hbm, out_hbm,
              idx0, idx1, sg0, sg1, rows0, rows1, ob0, ob1, pos_v, segtab_v,
              gsem0, gsem1, psem):
    cid = lax.axis_index("c")
    sid = lax.axis_index("s")
    wid = sid * NC + cid
    wbase = wid * BPW * L

    idx = (idx0, idx1)
    sg = (sg0, sg1)
    rows = (rows0, rows1)
    obuf = (ob0, ob1)
    gsem = (gsem0, gsem1)

    pltpu.sync_copy(segtab_hbm, segtab_v)
    r0 = [segtab_v[0, pl.ds(j * LANES, LANES)] for j in range(EV)]
    d1 = [segtab_v[1, pl.ds(j * LANES, LANES)] - r0[j] for j in range(EV)]
    d2 = [segtab_v[2, pl.ds(j * LANES, LANES)] - r0[j] for j in range(EV)]

    def fetch_and_gather(c, k, lcbase):
        base = pl.multiple_of(lcbase + c * L, CL)
        pltpu.sync_copy(seq_hbm.at[pl.ds(base, CL)], idx[k])
        pltpu.sync_copy(seg_hbm.at[pl.ds(base, CL)], sg[k])
        pltpu.async_copy(tok_hbm.at[idx[k]], rows[k], gsem[k])

    def compute_and_write(c, k, lcbase, pbase):
        base = pl.multiple_of(lcbase + c * L, CL)
        pltpu.make_async_copy(tok_hbm.at[idx[k]], rows[k], gsem[k]).wait()

        @plsc.parallel_loop(0, CL // LANES, 1, unroll=2)
        def group_body(g):
            i0 = pl.multiple_of(g * LANES, LANES)
            segf = sg[k][pl.ds(i0, LANES)].astype(jnp.float32)
            m1v = segf * (2.0 - segf)
            m2v = segf * (segf - 1.0) * 0.5
            for kk in range(LANES):
                m1 = jnp.broadcast_to(m1v[kk], (LANES,))
                m2 = jnp.broadcast_to(m2v[kk], (LANES,))
                r = i0 + kk
                pr = pbase + i0 + kk
                for j in range(EV):
                    sl = pl.ds(j * LANES, LANES)
                    obuf[k][r, sl] = (rows[k][r, sl] + pos_v[pr, sl]
                                      + d1[j] * m1 + d2[j] * m2)
        pltpu.sync_copy(obuf[k], out_hbm.at[pl.ds(base, CL)])

    pltpu.async_copy(pos_hbm.at[pl.ds(0, CL)], pos_v.at[pl.ds(0, CL)], psem)

    def lc_body(lc, _):
        lcbase = wbase + lc * CL
        pbase = pl.multiple_of(lax.rem(lc, 2) * CL, CL)
        pltpu.make_async_copy(pos_hbm.at[pl.ds(0, CL)],
                              pos_v.at[pl.ds(pbase, CL)], psem).wait()

        @plsc.parallel_loop(0, CL, 1, unroll=2)
        def pos_body(i):
            for j in range(EV):
                sl = pl.ds(j * LANES, LANES)
                pos_v[pbase + i, sl] = pos_v[pbase + i, sl] + r0[j]

        fetch_and_gather(0, 0, lcbase)

        def pair_body(t, _):
            for k in (0, 1):
                c = t * 2 + k

                @pl.when(c + 1 < BPW)
                def _():
                    fetch_and_gather(c + 1, k ^ 1, lcbase)

                compute_and_write(c, k, lcbase, pbase)
            return 0

        lax.fori_loop(0, BPW // 2, pair_body, 0)

        @pl.when(lc + 1 < NLC)
        def _():
            nb = pl.multiple_of(lax.rem(lc + 1, 2) * CL, CL)
            pltpu.async_copy(pos_hbm.at[pl.ds((lc + 1) * CL, CL)],
                             pos_v.at[pl.ds(nb, CL)], psem)

        return 0

    lax.fori_loop(0, NLC, lc_body, 0)


@functools.partial(jax.jit, static_argnames=())
def kernel(sequence, segment_label, token_table, position_table,
           segment_table):
    seq = sequence.reshape(-1).astype(jnp.int32)
    seg = segment_label.reshape(-1).astype(jnp.int32)

    mesh = plsc.VectorSubcoreMesh(core_axis_name="c", subcore_axis_name="s",
                                  num_cores=NC, num_subcores=NS)
    out = pl.kernel(
        _emb_body,
        out_type=jax.ShapeDtypeStruct((B * L, E), jnp.float32),
        mesh=mesh,
        scratch_types=[
            pltpu.VMEM((CL,), jnp.int32),
            pltpu.VMEM((CL,), jnp.int32),
            pltpu.VMEM((CL,), jnp.int32),
            pltpu.VMEM((CL,), jnp.int32),
            pltpu.VMEM((CL, E), jnp.float32),
            pltpu.VMEM((CL, E), jnp.float32),
            pltpu.VMEM((CL, E), jnp.float32),
            pltpu.VMEM((CL, E), jnp.float32),
            pltpu.VMEM((2 * CL, E), jnp.float32),
            pltpu.VMEM((3, E), jnp.float32),
            pltpu.SemaphoreType.DMA,
            pltpu.SemaphoreType.DMA,
            pltpu.SemaphoreType.DMA,
        ],
    )(seq, seg, token_table, position_table, segment_table)
    return out.reshape(B, L, E)


# speedup vs baseline: 2.7049x; 2.7049x over previous
import functools

import jax
import jax.numpy as jnp
from jax import lax
from jax.experimental import pallas as pl
from jax.experimental.pallas import tpu as pltpu
from jax.experimental.pallas import tpu_sc as plsc

B = 1024
L = 512
E = 128
VOCAB = 100000

NC = 2
NS = 16
NW = NC * NS
BPW = B // NW
CL = 128
NLC = L // CL
LANES = 16
EV = E // LANES


def _emb_body(seq_hbm, seg_hbm, tok_hbm, pos_hbm, segtab_hbm, out_hbm,
              idx0, idx1, sg0, sg1, rows0, rows1, pos_v, segtab_v,
              gsem0, gsem1):
    cid = lax.axis_index("c")
    sid = lax.axis_index("s")
    wid = sid * NC + cid
    wbase = wid * BPW * L

    idx = (idx0, idx1)
    sg = (sg0, sg1)
    rows = (rows0, rows1)
    gsem = (gsem0, gsem1)

    pltpu.sync_copy(segtab_hbm, segtab_v)
    r0 = [segtab_v[0, pl.ds(j * LANES, LANES)] for j in range(EV)]
    d1 = [segtab_v[1, pl.ds(j * LANES, LANES)] - r0[j] for j in range(EV)]
    d2 = [segtab_v[2, pl.ds(j * LANES, LANES)] - r0[j] for j in range(EV)]

    for lc in range(NLC):
        pltpu.sync_copy(pos_hbm.at[pl.ds(lc * CL, CL)], pos_v)

        def pos_body(i, _):
            for j in range(EV):
                sl = pl.ds(j * LANES, LANES)
                pos_v[i, sl] = pos_v[i, sl] + r0[j]
            return 0

        lax.fori_loop(0, CL, pos_body, 0)

        def fetch_and_gather(c, k, lc=lc):
            base = pl.multiple_of(wbase + lc * CL + c * L, CL)
            pltpu.sync_copy(seq_hbm.at[pl.ds(base, CL)], idx[k])
            pltpu.sync_copy(seg_hbm.at[pl.ds(base, CL)], sg[k])
            pltpu.async_copy(tok_hbm.at[idx[k]], rows[k], gsem[k])

        def compute_and_write(c, k, lc=lc):
            base = pl.multiple_of(wbase + lc * CL + c * L, CL)
            pltpu.make_async_copy(tok_hbm.at[idx[k]], rows[k],
                                  gsem[k]).wait()

            def group_body(g, _):
                i0 = pl.multiple_of(g * LANES, LANES)
                segf = sg[k][pl.ds(i0, LANES)].astype(jnp.float32)
                m1v = segf * (2.0 - segf)
                m2v = segf * (segf - 1.0) * 0.5
                for kk in range(LANES):
                    m1 = jnp.broadcast_to(m1v[kk], (LANES,))
                    m2 = jnp.broadcast_to(m2v[kk], (LANES,))
                    r = i0 + kk
                    for j in range(EV):
                        sl = pl.ds(j * LANES, LANES)
                        rows[k][r, sl] = (rows[k][r, sl] + pos_v[r, sl]
                                          + d1[j] * m1 + d2[j] * m2)
                return 0

            lax.fori_loop(0, CL // LANES, group_body, 0)
            pltpu.sync_copy(rows[k], out_hbm.at[pl.ds(base, CL)])

        fetch_and_gather(0, 0)

        def pair_body(t, _):
            for k in (0, 1):
                c = t * 2 + k

                @pl.when(c + 1 < BPW)
                def _():
                    fetch_and_gather(c + 1, k ^ 1)

                compute_and_write(c, k)
            return 0

        lax.fori_loop(0, BPW // 2, pair_body, 0)


@functools.partial(jax.jit, static_argnames=())
def kernel(sequence, segment_label, token_table, position_table,
           segment_table):
    seq = sequence.reshape(-1).astype(jnp.int32)
    seg = segment_label.reshape(-1).astype(jnp.int32)

    mesh = plsc.VectorSubcoreMesh(core_axis_name="c", subcore_axis_name="s",
                                  num_cores=NC, num_subcores=NS)
    out = pl.kernel(
        _emb_body,
        out_type=jax.ShapeDtypeStruct((B * L, E), jnp.float32),
        mesh=mesh,
        scratch_types=[
            pltpu.VMEM((CL,), jnp.int32),
            pltpu.VMEM((CL,), jnp.int32),
            pltpu.VMEM((CL,), jnp.int32),
            pltpu.VMEM((CL,), jnp.int32),
            pltpu.VMEM((CL, E), jnp.float32),
            pltpu.VMEM((CL, E), jnp.float32),
            pltpu.VMEM((CL, E), jnp.float32),
            pltpu.VMEM((3, E), jnp.float32),
            pltpu.SemaphoreType.DMA,
            pltpu.SemaphoreType.DMA,
        ],
    )(seq, seg, token_table, position_table, segment_table)
    return out.reshape(B, L, E)
